# baseline (speedup 1.0000x reference)
import jax
import jax.numpy as jnp
from jax.experimental import pallas as pl
from jax.experimental.pallas import tpu as pltpu

_B, _C, _H, _W = 4, 384, 32, 32
_HW = _H * _W
_H2, _W2 = 16, 16
_HW2 = _H2 * _W2


def _dot(a, b, dims):
    return jax.lax.dot_general(a, b, (dims, ((), ())),
                               preferred_element_type=jnp.float32)


def _sim_kernel(fp_ref, fms_ref, rt_ref, sums_ref, listk_ref):
    iota_c = jax.lax.broadcasted_iota(jnp.int32, (_C, _C), 1)
    fp = fp_ref[0]
    fms = fms_ref[0]
    p_d = fp - jnp.mean(fp, axis=1, keepdims=True)
    p_n = jnp.sqrt(jnp.sum(p_d * p_d, axis=1, keepdims=True))
    ms_res = _dot(fms, rt_ref[...], ((1,), (0,)))
    ms_mean = jnp.mean(ms_res, axis=1, keepdims=True)
    msq = jnp.sum(ms_res * ms_res, axis=1, keepdims=True)
    m_n = jnp.sqrt(msq - _HW * ms_mean * ms_mean)
    denom = m_n * p_n * 0.01
    s = _dot(ms_res, p_d, ((1,), (1,))) / denom
    mv = jnp.max(s, axis=1, keepdims=True)
    idx = jnp.min(jnp.where(s == mv, iota_c, _C), axis=1, keepdims=True)
    e = jnp.exp(mv - jnp.max(mv))
    max_val = e / jnp.sum(e)
    hits = idx == iota_c
    sums_ref[0] = jnp.sum(jnp.where(hits, max_val, 0.0), axis=0, keepdims=True)
    present = jnp.sum(hits.astype(jnp.int32), axis=0, keepdims=True) > 0
    listk_ref[0] = jnp.full((1, 128), jnp.sum(present.astype(jnp.int32)),
                            dtype=jnp.int32)


def _sel_kernel(fp_ref, sums_ref, listk_ref, out_ref):
    iota_r = jax.lax.broadcasted_iota(jnp.int32, (_C, _C), 0)
    iota_c = jax.lax.broadcasted_iota(jnp.int32, (_C, _C), 1)
    min_k = (jnp.min(listk_ref[...]) + 1) // 2
    sums = sums_ref[0]
    scol = jnp.transpose(sums)
    before = (sums > scol) | ((sums == scol) & (iota_c > iota_r))
    rank = jnp.sum(before.astype(jnp.int32), axis=1, keepdims=True)
    selm = rank < min_k
    mx = jnp.max(sums)
    w = jnp.where(selm, jnp.exp(scol - mx), 0.0)
    w = w / jnp.sum(w)
    fp = fp_ref[0]
    sig = jax.nn.sigmoid(fp)
    maskv = _dot(jnp.transpose(w), sig, ((1,), (0,)))
    out_ref[0] = fp * (1.0 + maskv)


def kernel(f_p, f_ms):
    B, C, H, W = f_p.shape
    a = jax.image.resize(jnp.eye(_H2, dtype=jnp.float32), (_H, _H2),
                         method="bilinear")
    rt = jnp.transpose(jnp.kron(a, a))
    fp_flat = f_p.reshape(B, C, H * W)
    fms_flat = f_ms.reshape(B, C, _HW2)

    sums, listk = pl.pallas_call(
        _sim_kernel,
        grid=(B,),
        out_shape=(
            jax.ShapeDtypeStruct((B, 1, C), jnp.float32),
            jax.ShapeDtypeStruct((B, 1, 128), jnp.int32),
        ),
        in_specs=[
            pl.BlockSpec((1, C, H * W), lambda b: (b, 0, 0)),
            pl.BlockSpec((1, C, _HW2), lambda b: (b, 0, 0)),
            pl.BlockSpec((_HW2, H * W), lambda b: (0, 0)),
        ],
        out_specs=(
            pl.BlockSpec((1, 1, C), lambda b: (b, 0, 0)),
            pl.BlockSpec((1, 1, 128), lambda b: (b, 0, 0)),
        ),
        compiler_params=pltpu.CompilerParams(
            dimension_semantics=("parallel",)),
    )(fp_flat, fms_flat, rt)

    out = pl.pallas_call(
        _sel_kernel,
        grid=(B,),
        out_shape=jax.ShapeDtypeStruct((B, C, H * W), jnp.float32),
        in_specs=[
            pl.BlockSpec((1, C, H * W), lambda b: (b, 0, 0)),
            pl.BlockSpec((1, 1, C), lambda b: (b, 0, 0)),
            pl.BlockSpec((B, 1, 128), lambda b: (0, 0, 0)),
        ],
        out_specs=pl.BlockSpec((1, C, H * W), lambda b: (b, 0, 0)),
        compiler_params=pltpu.CompilerParams(
            dimension_semantics=("parallel",)),
    )(fp_flat, sums, listk)
    return out.reshape(B, C, H, W)


# speedup vs baseline: 1.0803x; 1.0803x over previous
import jax
import jax.numpy as jnp
from jax.experimental import pallas as pl
from jax.experimental.pallas import tpu as pltpu

_B, _C, _H, _W = 4, 384, 32, 32
_HW = _H * _W
_H2, _W2 = 16, 16
_HW2 = _H2 * _W2


def _dot(a, b, dims):
    return jax.lax.dot_general(a, b, (dims, ((), ())),
                               preferred_element_type=jnp.float32)


def _mu_kernel(fp_ref, fms_ref, rt_ref, out_ref,
               fp_scr, sums_scr, listk_scr):
    i = pl.program_id(0)

    @pl.when(i < _B)
    def _sim():
        iota_c = jax.lax.broadcasted_iota(jnp.int32, (_C, _C), 1)
        fp = fp_ref[0]
        fms = fms_ref[0]
        fp_scr[i] = fp
        p_d = fp - jnp.mean(fp, axis=1, keepdims=True)
        p_n = jnp.sqrt(jnp.sum(p_d * p_d, axis=1, keepdims=True))
        ms_res = _dot(fms, rt_ref[...], ((1,), (0,)))
        ms_mean = jnp.mean(ms_res, axis=1, keepdims=True)
        msq = jnp.sum(ms_res * ms_res, axis=1, keepdims=True)
        m_n = jnp.sqrt(msq - _HW * ms_mean * ms_mean)
        denom = m_n * p_n * 0.01
        s = _dot(ms_res, p_d, ((1,), (1,))) / denom
        mv = jnp.max(s, axis=1, keepdims=True)
        idx = jnp.min(jnp.where(s == mv, iota_c, _C), axis=1, keepdims=True)
        e = jnp.exp(mv - jnp.max(mv))
        max_val = e / jnp.sum(e)
        hits = idx == iota_c
        sums_scr[i] = jnp.sum(jnp.where(hits, max_val, 0.0),
                              axis=0, keepdims=True)
        present = jnp.sum(hits.astype(jnp.int32), axis=0, keepdims=True) > 0
        listk_scr[i] = jnp.sum(present.astype(jnp.int32))

    @pl.when(i >= _B)
    def _sel():
        b = i - _B
        iota_r = jax.lax.broadcasted_iota(jnp.int32, (_C, _C), 0)
        iota_c = jax.lax.broadcasted_iota(jnp.int32, (_C, _C), 1)
        min_lk = listk_scr[0]
        for k in range(1, _B):
            min_lk = jnp.minimum(min_lk, listk_scr[k])
        min_k = (min_lk + 1) // 2
        sums = sums_scr[b]
        scol = jnp.transpose(sums)
        before = (sums > scol) | ((sums == scol) & (iota_c > iota_r))
        rank = jnp.sum(before.astype(jnp.int32), axis=1, keepdims=True)
        selm = rank < min_k
        mx = jnp.max(sums)
        w = jnp.where(selm, jnp.exp(scol - mx), 0.0)
        w = w / jnp.sum(w)
        fp = fp_scr[b]
        sig = jax.nn.sigmoid(fp)
        maskv = _dot(jnp.transpose(w), sig, ((1,), (0,)))
        out_ref[0] = fp * (1.0 + maskv)


def kernel(f_p, f_ms):
    B, C, H, W = f_p.shape
    a = jax.image.resize(jnp.eye(_H2, dtype=jnp.float32), (_H, _H2),
                         method="bilinear")
    rt = jnp.transpose(jnp.kron(a, a))
    fp_flat = f_p.reshape(B, C, H * W)
    fms_flat = f_ms.reshape(B, C, _HW2)

    out = pl.pallas_call(
        _mu_kernel,
        grid=(2 * B,),
        out_shape=jax.ShapeDtypeStruct((B, C, H * W), jnp.float32),
        in_specs=[
            pl.BlockSpec((1, C, H * W),
                         lambda i: (jnp.minimum(i, _B - 1), 0, 0)),
            pl.BlockSpec((1, C, _HW2),
                         lambda i: (jnp.minimum(i, _B - 1), 0, 0)),
            pl.BlockSpec((_HW2, H * W), lambda i: (0, 0)),
        ],
        out_specs=pl.BlockSpec((1, C, H * W),
                               lambda i: (jnp.maximum(i - _B, 0), 0, 0)),
        scratch_shapes=[
            pltpu.VMEM((B, C, H * W), jnp.float32),
            pltpu.VMEM((B, 1, C), jnp.float32),
            pltpu.SMEM((B,), jnp.int32),
        ],
    )(fp_flat, fms_flat, rt)
    return out.reshape(B, C, H, W)
